# Initial kernel scaffold; baseline (speedup 1.0000x reference)
import jax
import jax.numpy as jnp
from jax.experimental import pallas as pl


def kernel(indices, embedding_weight, row_mask):
    raise NotImplementedError("write your pallas kernel here")


# speedup vs baseline: 1.3474x; 1.3474x over previous
import functools

import jax
import jax.numpy as jnp
from jax import lax
from jax.experimental import pallas as pl
from jax.experimental.pallas import tpu as pltpu
from jax.experimental.pallas import tpu_sc as plsc

VOCAB = 1000000
DIM = 32
BATCH = 4096
HIST = 200

NC = 2
NS = 16
NW = NC * NS
LANES = 16

TOTAL = BATCH * HIST
GROUP = 128
GROUPS_PER_W = TOTAL // (NW * GROUP)


def _sc_body(idx_hbm, table_hbm, mask_hbm, out_hbm,
             idx_v, rows_v, mask_v, sem_idx, sem_rows, sem_mask):
    wid = lax.axis_index("s") * NC + lax.axis_index("c")
    pltpu.async_copy(idx_hbm.at[pl.ds(wid * GROUPS_PER_W, GROUPS_PER_W)],
                     idx_v, sem_idx).wait()

    def group_body(g, _):
        cp_rows = pltpu.async_copy(table_hbm.at[idx_v.at[g]], rows_v, sem_rows)
        cp_mask = pltpu.async_copy(mask_hbm.at[idx_v.at[g]], mask_v, sem_mask)
        cp_rows.wait()
        cp_mask.wait()

        def block16_body(k, _):
            mvec = mask_v[pl.ds(k * LANES, LANES)]
            for j in range(LANES):
                m = mvec[j]
                r = k * LANES + j
                rows_v[r, pl.ds(0, LANES)] = rows_v[r, pl.ds(0, LANES)] * m
                rows_v[r, pl.ds(LANES, LANES)] = rows_v[r, pl.ds(LANES, LANES)] * m
            return 0

        lax.fori_loop(0, GROUP // LANES, block16_body, 0)

        base = (wid * GROUPS_PER_W + g) * GROUP
        pltpu.async_copy(rows_v, out_hbm.at[pl.ds(base, GROUP)], sem_rows).wait()
        return 0

    lax.fori_loop(0, GROUPS_PER_W, group_body, 0)


@jax.jit
def _run(idx_flat, table, mask):
    mesh = plsc.VectorSubcoreMesh(core_axis_name="c", subcore_axis_name="s")
    fn = pl.kernel(
        _sc_body,
        out_type=jax.ShapeDtypeStruct((TOTAL, DIM), jnp.float32),
        mesh=mesh,
        scratch_types=[
            pltpu.VMEM((GROUPS_PER_W, GROUP), jnp.int32),
            pltpu.VMEM((GROUP, DIM), jnp.float32),
            pltpu.VMEM((GROUP,), jnp.float32),
            pltpu.SemaphoreType.DMA,
            pltpu.SemaphoreType.DMA,
            pltpu.SemaphoreType.DMA,
        ],
        compiler_params=pltpu.CompilerParams(use_tc_tiling_on_sc=False),
    )
    return fn(idx_flat, table, mask)


def kernel(indices, embedding_weight, row_mask):
    idx_flat = indices.reshape(TOTAL // GROUP, GROUP).astype(jnp.int32)
    out = _run(idx_flat, embedding_weight, row_mask.reshape(VOCAB))
    return out.reshape(BATCH, HIST, DIM)


# speedup vs baseline: 1.5656x; 1.1620x over previous
import jax
import jax.numpy as jnp
from jax import lax
from jax.experimental import pallas as pl
from jax.experimental.pallas import tpu as pltpu
from jax.experimental.pallas import tpu_sc as plsc

VOCAB = 1000000
DIM = 32
BATCH = 4096
HIST = 200

NC = 2
NS = 16
NW = NC * NS
LANES = 16

TOTAL = BATCH * HIST
GROUP = 128
GROUPS_PER_W = TOTAL // (NW * GROUP)
NBUF = 4


def _sc_body(idx_hbm, table_hbm, mask_hbm, out_hbm,
             idx_v, rows_v, mask_v, sems_g, sems_w, sem_idx):
    wid = lax.axis_index("s") * NC + lax.axis_index("c")
    pltpu.async_copy(idx_hbm.at[pl.ds(wid * GROUPS_PER_W, GROUPS_PER_W)],
                     idx_v, sem_idx).wait()

    def issue_gathers(g, b):
        pltpu.async_copy(table_hbm.at[idx_v.at[g]], rows_v.at[b], sems_g[b])
        pltpu.async_copy(mask_hbm.at[idx_v.at[g]], mask_v.at[b], sems_g[b])

    def wait_gathers(g, b):
        pltpu.make_async_copy(table_hbm.at[idx_v.at[g]], rows_v.at[b],
                              sems_g[b]).wait()
        pltpu.make_async_copy(mask_hbm.at[idx_v.at[g]], mask_v.at[b],
                              sems_g[b]).wait()

    def out_slice(g):
        return out_hbm.at[pl.ds((wid * GROUPS_PER_W + g) * GROUP, GROUP)]

    def issue_writeout(g, b):
        pltpu.async_copy(rows_v.at[b], out_slice(g), sems_w[b])

    def wait_writeout(g, b):
        pltpu.make_async_copy(rows_v.at[b], out_slice(g), sems_w[b]).wait()

    def compute(b):
        def block16_body(k, _):
            mvec = mask_v[b, pl.ds(k * LANES, LANES)]
            for j in range(LANES):
                m = mvec[j]
                r = k * LANES + j
                rows_v[b, r, pl.ds(0, LANES)] = rows_v[b, r, pl.ds(0, LANES)] * m
                rows_v[b, r, pl.ds(LANES, LANES)] = (
                    rows_v[b, r, pl.ds(LANES, LANES)] * m)
            return 0

        lax.fori_loop(0, GROUP // LANES, block16_body, 0)

    issue_gathers(0, 0)
    issue_gathers(1, 1)
    for g in (0, 1):
        issue_gathers(g + 2, g + 2)
        wait_gathers(g, g)
        compute(g)
        issue_writeout(g, g)

    def quad_body(g2, _):
        for b in range(NBUF):
            g = 2 + g2 * NBUF + b
            bb = (2 + b) % NBUF
            bp = b % NBUF
            wait_writeout(g - 2, bp)
            issue_gathers(g + 2, bp)
            wait_gathers(g, bb)
            compute(bb)
            issue_writeout(g, bb)
        return 0

    lax.fori_loop(0, (GROUPS_PER_W - 4) // NBUF, quad_body, 0)

    for g in (GROUPS_PER_W - 2, GROUPS_PER_W - 1):
        b = g % NBUF
        wait_gathers(g, b)
        compute(b)
        issue_writeout(g, b)
    for g in range(GROUPS_PER_W - 4, GROUPS_PER_W):
        wait_writeout(g, g % NBUF)


@jax.jit
def _run(idx_flat, table, mask):
    mesh = plsc.VectorSubcoreMesh(core_axis_name="c", subcore_axis_name="s")
    fn = pl.kernel(
        _sc_body,
        out_type=jax.ShapeDtypeStruct((TOTAL, DIM), jnp.float32),
        mesh=mesh,
        scratch_types=[
            pltpu.VMEM((GROUPS_PER_W, GROUP), jnp.int32),
            pltpu.VMEM((NBUF, GROUP, DIM), jnp.float32),
            pltpu.VMEM((NBUF, GROUP), jnp.float32),
            [pltpu.SemaphoreType.DMA] * NBUF,
            [pltpu.SemaphoreType.DMA] * NBUF,
            pltpu.SemaphoreType.DMA,
        ],
        compiler_params=pltpu.CompilerParams(use_tc_tiling_on_sc=False),
    )
    return fn(idx_flat, table, mask)


def kernel(indices, embedding_weight, row_mask):
    idx_flat = indices.reshape(TOTAL // GROUP, GROUP).astype(jnp.int32)
    out = _run(idx_flat, embedding_weight, row_mask.reshape(VOCAB))
    return out.reshape(BATCH, HIST, DIM)


# speedup vs baseline: 1.5876x; 1.0140x over previous
import jax
import jax.numpy as jnp
from jax import lax
from jax.experimental import pallas as pl
from jax.experimental.pallas import tpu as pltpu
from jax.experimental.pallas import tpu_sc as plsc

VOCAB = 1000000
DIM = 32
BATCH = 4096
HIST = 200

NC = 2
NS = 16
NW = NC * NS
LANES = 16

TOTAL = BATCH * HIST
GROUP = 128
GROUPS_PER_W = TOTAL // (NW * GROUP)
NBUF = 8
DEPTH = 4


def _sc_body(idx_hbm, table_hbm, mask_hbm, out_hbm,
             idx_v, rows_v, mask_v, sems_g, sems_w, sem_idx):
    wid = lax.axis_index("s") * NC + lax.axis_index("c")
    pltpu.async_copy(idx_hbm.at[pl.ds(wid * GROUPS_PER_W, GROUPS_PER_W)],
                     idx_v, sem_idx).wait()

    def issue_gathers(g, b):
        pltpu.async_copy(table_hbm.at[idx_v.at[g]], rows_v.at[b], sems_g[b])
        pltpu.async_copy(mask_hbm.at[idx_v.at[g]], mask_v.at[b], sems_g[b])

    def wait_gathers(g, b):
        pltpu.make_async_copy(table_hbm.at[idx_v.at[g]], rows_v.at[b],
                              sems_g[b]).wait()
        pltpu.make_async_copy(mask_hbm.at[idx_v.at[g]], mask_v.at[b],
                              sems_g[b]).wait()

    def out_slice(g):
        return out_hbm.at[pl.ds((wid * GROUPS_PER_W + g) * GROUP, GROUP)]

    def issue_writeout(g, b):
        pltpu.async_copy(rows_v.at[b], out_slice(g), sems_w[b])

    def wait_writeout(g, b):
        pltpu.make_async_copy(rows_v.at[b], out_slice(g), sems_w[b]).wait()

    def compute(b):
        def block16_body(k, _):
            mvec = mask_v[b, pl.ds(k * LANES, LANES)]
            for j in range(LANES):
                m = mvec[j]
                r = k * LANES + j
                rows_v[b, r, pl.ds(0, LANES)] = rows_v[b, r, pl.ds(0, LANES)] * m
                rows_v[b, r, pl.ds(LANES, LANES)] = (
                    rows_v[b, r, pl.ds(LANES, LANES)] * m)
            return 0

        lax.fori_loop(0, GROUP // LANES, block16_body, 0)

    for b in range(DEPTH):
        issue_gathers(b, b)
    for g in range(DEPTH):
        issue_gathers(g + DEPTH, g + DEPTH)
        wait_gathers(g, g)
        compute(g)
        issue_writeout(g, g)

    def oct_body(g2, _):
        for b in range(NBUF):
            g = DEPTH + g2 * NBUF + b
            bb = (DEPTH + b) % NBUF
            bp = b % NBUF
            wait_writeout(g - DEPTH, bp)
            issue_gathers(g + DEPTH, bp)
            wait_gathers(g, bb)
            compute(bb)
            issue_writeout(g, bb)
        return 0

    lax.fori_loop(0, (GROUPS_PER_W - 2 * DEPTH) // NBUF, oct_body, 0)

    for g in range(GROUPS_PER_W - DEPTH, GROUPS_PER_W):
        b = g % NBUF
        wait_gathers(g, b)
        compute(b)
        issue_writeout(g, b)
    for g in range(GROUPS_PER_W - 2 * DEPTH, GROUPS_PER_W):
        wait_writeout(g, g % NBUF)


@jax.jit
def _run(idx_flat, table, mask):
    mesh = plsc.VectorSubcoreMesh(core_axis_name="c", subcore_axis_name="s")
    fn = pl.kernel(
        _sc_body,
        out_type=jax.ShapeDtypeStruct((TOTAL, DIM), jnp.float32),
        mesh=mesh,
        scratch_types=[
            pltpu.VMEM((GROUPS_PER_W, GROUP), jnp.int32),
            pltpu.VMEM((NBUF, GROUP, DIM), jnp.float32),
            pltpu.VMEM((NBUF, GROUP), jnp.float32),
            [pltpu.SemaphoreType.DMA] * NBUF,
            [pltpu.SemaphoreType.DMA] * NBUF,
            pltpu.SemaphoreType.DMA,
        ],
        compiler_params=pltpu.CompilerParams(use_tc_tiling_on_sc=False),
    )
    return fn(idx_flat, table, mask)


def kernel(indices, embedding_weight, row_mask):
    idx_flat = indices.reshape(TOTAL // GROUP, GROUP).astype(jnp.int32)
    out = _run(idx_flat, embedding_weight, row_mask.reshape(VOCAB))
    return out.reshape(BATCH, HIST, DIM)


# speedup vs baseline: 1.5879x; 1.0002x over previous
import jax
import jax.numpy as jnp
from jax import lax
from jax.experimental import pallas as pl
from jax.experimental.pallas import tpu as pltpu
from jax.experimental.pallas import tpu_sc as plsc

VOCAB = 1000000
DIM = 32
BATCH = 4096
HIST = 200

NC = 2
NS = 16
NW = NC * NS
LANES = 16

TOTAL = BATCH * HIST
GROUP = 256
GROUPS_PER_W = TOTAL // (NW * GROUP)
NBUF = 8
DEPTH = 4


def _sc_body(idx_hbm, table_hbm, mask_hbm, out_hbm,
             idx_v, rows_v, mask_v, sems_g, sems_w, sem_idx):
    wid = lax.axis_index("s") * NC + lax.axis_index("c")
    pltpu.async_copy(idx_hbm.at[pl.ds(wid * GROUPS_PER_W, GROUPS_PER_W)],
                     idx_v, sem_idx).wait()

    def issue_gathers(g, b):
        pltpu.async_copy(table_hbm.at[idx_v.at[g]], rows_v.at[b], sems_g[b])
        pltpu.async_copy(mask_hbm.at[idx_v.at[g]], mask_v.at[b], sems_g[b])

    def wait_gathers(g, b):
        pltpu.make_async_copy(table_hbm.at[idx_v.at[g]], rows_v.at[b],
                              sems_g[b]).wait()
        pltpu.make_async_copy(mask_hbm.at[idx_v.at[g]], mask_v.at[b],
                              sems_g[b]).wait()

    def out_slice(g):
        return out_hbm.at[pl.ds((wid * GROUPS_PER_W + g) * GROUP, GROUP)]

    def issue_writeout(g, b):
        pltpu.async_copy(rows_v.at[b], out_slice(g), sems_w[b])

    def wait_writeout(g, b):
        pltpu.make_async_copy(rows_v.at[b], out_slice(g), sems_w[b]).wait()

    def compute(b):
        def block16_body(k, _):
            mvec = mask_v[b, pl.ds(k * LANES, LANES)]
            for j in range(LANES):
                m = mvec[j]
                r = k * LANES + j
                rows_v[b, r, pl.ds(0, LANES)] = rows_v[b, r, pl.ds(0, LANES)] * m
                rows_v[b, r, pl.ds(LANES, LANES)] = (
                    rows_v[b, r, pl.ds(LANES, LANES)] * m)
            return 0

        lax.fori_loop(0, GROUP // LANES, block16_body, 0)

    for b in range(DEPTH):
        issue_gathers(b, b)
    for g in range(DEPTH):
        issue_gathers(g + DEPTH, g + DEPTH)
        wait_gathers(g, g)
        compute(g)
        issue_writeout(g, g)

    def oct_body(g2, _):
        for b in range(NBUF):
            g = DEPTH + g2 * NBUF + b
            bb = (DEPTH + b) % NBUF
            bp = b % NBUF
            wait_writeout(g - DEPTH, bp)
            issue_gathers(g + DEPTH, bp)
            wait_gathers(g, bb)
            compute(bb)
            issue_writeout(g, bb)
        return 0

    nsteady = ((GROUPS_PER_W - 2 * DEPTH) // NBUF) * NBUF
    lax.fori_loop(0, nsteady // NBUF, oct_body, 0)

    for g in range(DEPTH + nsteady, GROUPS_PER_W - DEPTH):
        bb = g % NBUF
        bp = (g + DEPTH) % NBUF
        wait_writeout(g - DEPTH, bp)
        issue_gathers(g + DEPTH, bp)
        wait_gathers(g, bb)
        compute(bb)
        issue_writeout(g, bb)

    for g in range(GROUPS_PER_W - DEPTH, GROUPS_PER_W):
        b = g % NBUF
        wait_gathers(g, b)
        compute(b)
        issue_writeout(g, b)
    for g in range(GROUPS_PER_W - 2 * DEPTH, GROUPS_PER_W):
        wait_writeout(g, g % NBUF)


@jax.jit
def _run(idx_flat, table, mask):
    mesh = plsc.VectorSubcoreMesh(core_axis_name="c", subcore_axis_name="s")
    fn = pl.kernel(
        _sc_body,
        out_type=jax.ShapeDtypeStruct((TOTAL, DIM), jnp.float32),
        mesh=mesh,
        scratch_types=[
            pltpu.VMEM((GROUPS_PER_W, GROUP), jnp.int32),
            pltpu.VMEM((NBUF, GROUP, DIM), jnp.float32),
            pltpu.VMEM((NBUF, GROUP), jnp.float32),
            [pltpu.SemaphoreType.DMA] * NBUF,
            [pltpu.SemaphoreType.DMA] * NBUF,
            pltpu.SemaphoreType.DMA,
        ],
        compiler_params=pltpu.CompilerParams(use_tc_tiling_on_sc=False),
    )
    return fn(idx_flat, table, mask)


def kernel(indices, embedding_weight, row_mask):
    idx_flat = indices.reshape(TOTAL // GROUP, GROUP).astype(jnp.int32)
    out = _run(idx_flat, embedding_weight, row_mask.reshape(VOCAB))
    return out.reshape(BATCH, HIST, DIM)
